# Initial kernel scaffold; baseline (speedup 1.0000x reference)
import jax
import jax.numpy as jnp
from jax.experimental import pallas as pl


def kernel(x, offsets, emb_table, lin_w, lin_b, W1, b1, W2, b2, W3, b3):
    raise NotImplementedError("write your pallas kernel here")


# speedup vs baseline: 22.2245x; 22.2245x over previous
import functools

import jax
import jax.numpy as jnp
import numpy as np
from jax import lax
from jax.experimental import pallas as pl
from jax.experimental.pallas import tpu as pltpu
from jax.experimental.pallas import tpu_sc as plsc

F = 26
D = 16
B = 4096
TOTAL = 26000
NUM_INTER = F * (F - 1) // 2
NI_PAD = 336
ROW_W = F * D

_info = plsc.get_sparse_core_info()
NC, NS, L = _info.num_cores, _info.num_subcores, _info.num_lanes
NW = NC * NS
B_PER_W = B // NW
CHUNK = 4
N_CHUNKS = B_PER_W // CHUNK
N_GROUPS = NI_PAD // L
N_LIN_G = B_PER_W // L

_iu, _ju = np.triu_indices(F, k=1)
_r1 = np.concatenate([_iu, np.zeros(NI_PAD - NUM_INTER, np.int32)]).astype(np.int32)
_r2 = np.concatenate([_ju, np.zeros(NI_PAD - NUM_INTER, np.int32)]).astype(np.int32)
_c1 = _r2 * D
_c2 = _r1 * D
_PTAB_NP = np.concatenate([_r1, _c1, _r2, _c2])


def _sc_body(idx_hbm, ptab_hbm, linw_hbm, table_hbm, inter_hbm, lin_hbm,
             idx_v, ptab_v, linw_v, rows_v, out4_v, lin_v, sem):
    w = lax.axis_index("s") * NC + lax.axis_index("c")
    pltpu.sync_copy(idx_hbm.at[pl.ds(w * (B_PER_W * F), B_PER_W * F)], idx_v)
    pltpu.sync_copy(ptab_hbm, ptab_v)
    pltpu.sync_copy(linw_hbm, linw_v)

    def chunk_body(t, carry):
        pltpu.async_copy(
            table_hbm.at[idx_v.at[pl.ds(t * (CHUNK * F), CHUNK * F)]],
            rows_v, sem).wait()
        for k in range(CHUNK):
            def g_body(g, c):
                r1 = ptab_v[pl.ds(g * L, L)] + k * F
                c1 = ptab_v[pl.ds(NI_PAD + g * L, L)]
                r2 = ptab_v[pl.ds(2 * NI_PAD + g * L, L)] + k * F
                c2 = ptab_v[pl.ds(3 * NI_PAD + g * L, L)]
                acc = jnp.zeros((L,), jnp.float32)
                for d in range(D):
                    a = plsc.load_gather(rows_v, [r1, c1 + d])
                    b = plsc.load_gather(rows_v, [r2, c2 + d])
                    acc = acc + a * b
                out4_v[pl.ds(k * NI_PAD + g * L, L)] = acc
                return c
            lax.fori_loop(0, N_GROUPS, g_body, 0)
        pltpu.sync_copy(
            out4_v,
            inter_hbm.at[pl.ds((w * B_PER_W + t * CHUNK) * NI_PAD, CHUNK * NI_PAD)])
        return carry

    lax.fori_loop(0, N_CHUNKS, chunk_body, 0)

    io = lax.iota(jnp.int32, L) * F
    def lin_body(g, carry):
        acc = jnp.zeros((L,), jnp.float32)
        for f in range(F):
            rows_ids = plsc.load_gather(idx_v, [io + (g * (L * F) + f)])
            acc = acc + plsc.load_gather(linw_v, [rows_ids])
        lin_v[pl.ds(g * L, L)] = acc
        return carry
    lax.fori_loop(0, N_LIN_G, lin_body, 0)
    pltpu.sync_copy(lin_v, lin_hbm.at[pl.ds(w * B_PER_W, B_PER_W)])


_sc_call = functools.partial(
    pl.kernel,
    mesh=plsc.VectorSubcoreMesh(core_axis_name="c", subcore_axis_name="s"),
    compiler_params=pltpu.CompilerParams(
        use_tc_tiling_on_sc=False, needs_layout_passes=False),
    out_type=[
        jax.ShapeDtypeStruct((B * NI_PAD,), jnp.float32),
        jax.ShapeDtypeStruct((B,), jnp.float32),
    ],
    scratch_types=[
        pltpu.VMEM((B_PER_W * F,), jnp.int32),
        pltpu.VMEM((4 * NI_PAD,), jnp.int32),
        pltpu.VMEM((TOTAL,), jnp.float32),
        pltpu.VMEM((CHUNK * F, ROW_W), jnp.float32),
        pltpu.VMEM((CHUNK * NI_PAD,), jnp.float32),
        pltpu.VMEM((B_PER_W,), jnp.float32),
        pltpu.SemaphoreType.DMA,
    ],
)(_sc_body)


def _mlp_body(x_ref, w1_ref, b1_ref, w2_ref, b2_ref, w3_ref, b3_ref,
              lin_ref, linb_ref, o_ref):
    x = x_ref[...]
    h = jnp.maximum(
        jnp.dot(x, w1_ref[...], preferred_element_type=jnp.float32) + b1_ref[...], 0.0)
    h = jnp.maximum(
        jnp.dot(h, w2_ref[...], preferred_element_type=jnp.float32) + b2_ref[...], 0.0)
    o = jnp.dot(h, w3_ref[...], preferred_element_type=jnp.float32)
    o_ref[...] = o + b3_ref[...] + linb_ref[...] + lin_ref[...]


_MLP_BLK = 512


def _mlp_call(inter, w1p, b1, w2, b2, w3, b3, lin, lin_b):
    grid = (B // _MLP_BLK,)
    fixed = lambda i: (0, 0)
    return pl.pallas_call(
        _mlp_body,
        grid=grid,
        in_specs=[
            pl.BlockSpec((_MLP_BLK, NI_PAD), lambda i: (i, 0)),
            pl.BlockSpec((NI_PAD, 128), fixed),
            pl.BlockSpec((1, 128), fixed),
            pl.BlockSpec((128, 64), fixed),
            pl.BlockSpec((1, 64), fixed),
            pl.BlockSpec((64, 1), fixed),
            pl.BlockSpec((1, 1), fixed),
            pl.BlockSpec((_MLP_BLK, 1), lambda i: (i, 0)),
            pl.BlockSpec((1, 1), fixed),
        ],
        out_specs=pl.BlockSpec((_MLP_BLK, 1), lambda i: (i, 0)),
        out_shape=jax.ShapeDtypeStruct((B, 1), jnp.float32),
    )(inter, w1p, b1, w2, b2, w3, b3, lin, lin_b)


def kernel(x, offsets, emb_table, lin_w, lin_b, W1, b1, W2, b2, W3, b3):
    idx = (x.astype(jnp.int32) + offsets.astype(jnp.int32)[None, :]).reshape(-1)
    table2d = emb_table.reshape(TOTAL, ROW_W)
    ptab = jnp.asarray(_PTAB_NP, dtype=jnp.int32)
    inter_flat, lin_vec = _sc_call(idx, ptab, lin_w.reshape(-1), table2d)
    inter = inter_flat.reshape(B, NI_PAD)
    w1p = jnp.concatenate(
        [W1, jnp.zeros((NI_PAD - NUM_INTER, 128), jnp.float32)], axis=0)
    return _mlp_call(
        inter, w1p, b1.reshape(1, 128), W2, b2.reshape(1, 64), W3,
        b3.reshape(1, 1), lin_vec.reshape(B, 1), lin_b.reshape(1, 1))


# speedup vs baseline: 24.0641x; 1.0828x over previous
import functools

import jax
import jax.numpy as jnp
import numpy as np
from jax import lax
from jax.experimental import pallas as pl
from jax.experimental.pallas import tpu as pltpu
from jax.experimental.pallas import tpu_sc as plsc

F = 26
D = 16
B = 4096
TOTAL = 26000
NUM_INTER = F * (F - 1) // 2
NI_PAD = 336
ROW_W = F * D

_info = plsc.get_sparse_core_info()
NC, NS, L = _info.num_cores, _info.num_subcores, _info.num_lanes
NW = NC * NS
B_PER_W = B // NW
CHUNK = 4
N_CHUNKS = B_PER_W // CHUNK
N_GROUPS = NI_PAD // L
N_LIN_G = B_PER_W // L

_iu, _ju = np.triu_indices(F, k=1)
_r1 = np.concatenate([_iu, np.zeros(NI_PAD - NUM_INTER, np.int32)]).astype(np.int32)
_r2 = np.concatenate([_ju, np.zeros(NI_PAD - NUM_INTER, np.int32)]).astype(np.int32)
_c1 = _r2 * D
_c2 = _r1 * D
_PTAB_NP = np.concatenate([_r1, _c1, _r2, _c2])


def _sc_body(idx_hbm, ptab_hbm, linw_hbm, table_hbm, inter_hbm, lin_hbm,
             idx_v, ptab_v, linw_v, rows_a, rows_b, out4_v, lin_v,
             sem_a, sem_b):
    w = lax.axis_index("s") * NC + lax.axis_index("c")
    pltpu.sync_copy(idx_hbm.at[pl.ds(w * (B_PER_W * F), B_PER_W * F)], idx_v)
    pltpu.sync_copy(ptab_hbm, ptab_v)
    pltpu.sync_copy(linw_hbm, linw_v)

    def gather_start(t, rows, sem):
        pltpu.make_async_copy(
            table_hbm.at[idx_v.at[pl.ds(t * (CHUNK * F), CHUNK * F)]],
            rows, sem).start()

    def gather_wait(t, rows, sem):
        pltpu.make_async_copy(
            table_hbm.at[idx_v.at[pl.ds(t * (CHUNK * F), CHUNK * F)]],
            rows, sem).wait()

    def compute_chunk(t, rows_v):
        for k in range(CHUNK):
            def g_body(g, c):
                r1 = ptab_v[pl.ds(g * L, L)] + k * F
                c1 = ptab_v[pl.ds(NI_PAD + g * L, L)]
                r2 = ptab_v[pl.ds(2 * NI_PAD + g * L, L)] + k * F
                c2 = ptab_v[pl.ds(3 * NI_PAD + g * L, L)]
                a0 = plsc.load_gather(rows_v, [r1, c1])
                b0 = plsc.load_gather(rows_v, [r2, c2])
                a1 = plsc.load_gather(rows_v, [r1, c1 + 1])
                b1 = plsc.load_gather(rows_v, [r2, c2 + 1])
                acc0 = a0 * b0
                acc1 = a1 * b1
                for d in range(2, D, 2):
                    a0 = plsc.load_gather(rows_v, [r1, c1 + d])
                    b0 = plsc.load_gather(rows_v, [r2, c2 + d])
                    a1 = plsc.load_gather(rows_v, [r1, c1 + d + 1])
                    b1 = plsc.load_gather(rows_v, [r2, c2 + d + 1])
                    acc0 = acc0 + a0 * b0
                    acc1 = acc1 + a1 * b1
                out4_v[pl.ds(k * NI_PAD + g * L, L)] = acc0 + acc1
                return c
            lax.fori_loop(0, N_GROUPS, g_body, 0)
        pltpu.sync_copy(
            out4_v,
            inter_hbm.at[pl.ds((w * B_PER_W + t * CHUNK) * NI_PAD, CHUNK * NI_PAD)])

    gather_start(0, rows_a, sem_a)

    def pair_body(u, carry):
        t0 = 2 * u
        gather_start(t0 + 1, rows_b, sem_b)
        gather_wait(t0, rows_a, sem_a)
        compute_chunk(t0, rows_a)
        gather_start(jnp.bitwise_and(t0 + 2, N_CHUNKS - 1), rows_a, sem_a)
        gather_wait(t0 + 1, rows_b, sem_b)
        compute_chunk(t0 + 1, rows_b)
        return carry

    lax.fori_loop(0, N_CHUNKS // 2, pair_body, 0)
    gather_wait(0, rows_a, sem_a)

    io = lax.iota(jnp.int32, L) * F
    def lin_body(g, carry):
        acc = jnp.zeros((L,), jnp.float32)
        for f in range(F):
            rows_ids = plsc.load_gather(idx_v, [io + (g * (L * F) + f)])
            acc = acc + plsc.load_gather(linw_v, [rows_ids])
        lin_v[pl.ds(g * L, L)] = acc
        return carry
    lax.fori_loop(0, N_LIN_G, lin_body, 0)
    pltpu.sync_copy(lin_v, lin_hbm.at[pl.ds(w * B_PER_W, B_PER_W)])


_sc_call = functools.partial(
    pl.kernel,
    mesh=plsc.VectorSubcoreMesh(core_axis_name="c", subcore_axis_name="s"),
    compiler_params=pltpu.CompilerParams(
        use_tc_tiling_on_sc=False, needs_layout_passes=False),
    out_type=[
        jax.ShapeDtypeStruct((B * NI_PAD,), jnp.float32),
        jax.ShapeDtypeStruct((B,), jnp.float32),
    ],
    scratch_types=[
        pltpu.VMEM((B_PER_W * F,), jnp.int32),
        pltpu.VMEM((4 * NI_PAD,), jnp.int32),
        pltpu.VMEM((TOTAL,), jnp.float32),
        pltpu.VMEM((CHUNK * F, ROW_W), jnp.float32),
        pltpu.VMEM((CHUNK * F, ROW_W), jnp.float32),
        pltpu.VMEM((CHUNK * NI_PAD,), jnp.float32),
        pltpu.VMEM((B_PER_W,), jnp.float32),
        pltpu.SemaphoreType.DMA,
        pltpu.SemaphoreType.DMA,
    ],
)(_sc_body)


def _mlp_body(x_ref, w1_ref, b1_ref, w2_ref, b2_ref, w3_ref, b3_ref,
              lin_ref, linb_ref, o_ref):
    x = x_ref[...]
    h = jnp.maximum(
        jnp.dot(x, w1_ref[...], preferred_element_type=jnp.float32) + b1_ref[...], 0.0)
    h = jnp.maximum(
        jnp.dot(h, w2_ref[...], preferred_element_type=jnp.float32) + b2_ref[...], 0.0)
    o = jnp.dot(h, w3_ref[...], preferred_element_type=jnp.float32)
    o_ref[...] = o + b3_ref[...] + linb_ref[...] + lin_ref[...]


_MLP_BLK = 512


def _mlp_call(inter, w1p, b1, w2, b2, w3, b3, lin, lin_b):
    grid = (B // _MLP_BLK,)
    fixed = lambda i: (0, 0)
    return pl.pallas_call(
        _mlp_body,
        grid=grid,
        in_specs=[
            pl.BlockSpec((_MLP_BLK, NI_PAD), lambda i: (i, 0)),
            pl.BlockSpec((NI_PAD, 128), fixed),
            pl.BlockSpec((1, 128), fixed),
            pl.BlockSpec((128, 64), fixed),
            pl.BlockSpec((1, 64), fixed),
            pl.BlockSpec((64, 1), fixed),
            pl.BlockSpec((1, 1), fixed),
            pl.BlockSpec((_MLP_BLK, 1), lambda i: (i, 0)),
            pl.BlockSpec((1, 1), fixed),
        ],
        out_specs=pl.BlockSpec((_MLP_BLK, 1), lambda i: (i, 0)),
        out_shape=jax.ShapeDtypeStruct((B, 1), jnp.float32),
    )(inter, w1p, b1, w2, b2, w3, b3, lin, lin_b)


def kernel(x, offsets, emb_table, lin_w, lin_b, W1, b1, W2, b2, W3, b3):
    idx = (x.astype(jnp.int32) + offsets.astype(jnp.int32)[None, :]).reshape(-1)
    table2d = emb_table.reshape(TOTAL, ROW_W)
    ptab = jnp.asarray(_PTAB_NP, dtype=jnp.int32)
    inter_flat, lin_vec = _sc_call(idx, ptab, lin_w.reshape(-1), table2d)
    inter = inter_flat.reshape(B, NI_PAD)
    w1p = jnp.concatenate(
        [W1, jnp.zeros((NI_PAD - NUM_INTER, 128), jnp.float32)], axis=0)
    return _mlp_call(
        inter, w1p, b1.reshape(1, 128), W2, b2.reshape(1, 64), W3,
        b3.reshape(1, 1), lin_vec.reshape(B, 1), lin_b.reshape(1, 1))
